# baseline (speedup 1.0000x reference)
import jax
import jax.numpy as jnp
from jax.experimental import pallas as pl

_SZ = 2048
_TEMP = 10.0
_EPS = 1e-20
_BR = 256
_KC = 256


def _body(a_ref, u_ref, o_ref):
    l_idx = jax.lax.broadcasted_iota(jnp.int32, (_KC, _KC // 2), 0)
    k_idx = jax.lax.broadcasted_iota(jnp.int32, (_KC, _KC // 2), 1)
    d = jnp.where(l_idx == 2 * k_idx, 1.0, 0.0) - jnp.where(
        l_idx == 2 * k_idx + 1, 1.0, 0.0
    )
    w = (a_ref[...] - jnp.log(_EPS - jnp.log(u_ref[...] + _EPS))) * (1.0 / _TEMP)
    for c in range(2 * _SZ // _KC):
        x = jax.lax.dot_general(
            w[:, c * _KC : (c + 1) * _KC],
            d,
            (((1,), (0,)), ((), ())),
            precision=jax.lax.Precision.HIGHEST,
            preferred_element_type=jnp.float32,
        )
        o_ref[:, c * (_KC // 2) : (c + 1) * (_KC // 2)] = jax.nn.sigmoid(x)


def kernel(gen_matrix, u):
    a = gen_matrix.reshape(_SZ, 2 * _SZ)
    uu = u.reshape(_SZ, 2 * _SZ)
    return pl.pallas_call(
        _body,
        grid=(_SZ // _BR,),
        in_specs=[
            pl.BlockSpec((_BR, 2 * _SZ), lambda i: (i, 0)),
            pl.BlockSpec((_BR, 2 * _SZ), lambda i: (i, 0)),
        ],
        out_specs=pl.BlockSpec((_BR, _SZ), lambda i: (i, 0)),
        out_shape=jax.ShapeDtypeStruct((_SZ, _SZ), jnp.float32),
    )(a, uu)


# speedup vs baseline: 49.0932x; 49.0932x over previous
import jax
import jax.numpy as jnp
from jax.experimental import pallas as pl

_SZ = 2048
_TEMP = 10.0
_EPS = 1e-20
_BR = 256


def _body(d_ref, u0_ref, u1_ref, o_ref):
    l0 = _EPS - jnp.log(u0_ref[...] + _EPS)
    l1 = _EPS - jnp.log(u1_ref[...] + _EPS)
    x = (d_ref[...] + jnp.log(l1 / l0)) * (1.0 / _TEMP)
    o_ref[...] = jax.nn.sigmoid(x)


def kernel(gen_matrix, u):
    u3 = u.reshape(_SZ, _SZ, 2)
    d = gen_matrix[:, :, 0] - gen_matrix[:, :, 1]
    u0 = u3[:, :, 0]
    u1 = u3[:, :, 1]
    spec = pl.BlockSpec((_BR, _SZ), lambda i: (i, 0))
    return pl.pallas_call(
        _body,
        grid=(_SZ // _BR,),
        in_specs=[spec, spec, spec],
        out_specs=spec,
        out_shape=jax.ShapeDtypeStruct((_SZ, _SZ), jnp.float32),
    )(d, u0, u1)


# speedup vs baseline: 57.9071x; 1.1795x over previous
import jax
import jax.numpy as jnp
from jax.experimental import pallas as pl

_SZ = 2048
_TEMP = 10.0
_EPS = 1e-20
_BR = 256


def _body(d_ref, ut_ref, o_ref):
    l0 = _EPS - jnp.log(ut_ref[0] + _EPS)
    l1 = _EPS - jnp.log(ut_ref[1] + _EPS)
    x = (d_ref[...] + jnp.log(l1 / l0)) * (1.0 / _TEMP)
    o_ref[...] = jax.nn.sigmoid(x)


def kernel(gen_matrix, u):
    u3 = u.reshape(_SZ, _SZ, 2)
    ut = jnp.moveaxis(u3, 2, 0)
    d = gen_matrix[:, :, 0] - gen_matrix[:, :, 1]
    spec = pl.BlockSpec((_BR, _SZ), lambda i: (i, 0))
    return pl.pallas_call(
        _body,
        grid=(_SZ // _BR,),
        in_specs=[spec, pl.BlockSpec((2, _BR, _SZ), lambda i: (0, i, 0))],
        out_specs=spec,
        out_shape=jax.ShapeDtypeStruct((_SZ, _SZ), jnp.float32),
    )(d, ut)


# speedup vs baseline: 68.1138x; 1.1763x over previous
import jax
import jax.numpy as jnp
from jax.experimental import pallas as pl

_SZ = 2048
_TEMP = 10.0
_EPS = 1e-20
_BR = 256


def _body(at_ref, ut_ref, o_ref):
    l0 = _EPS - jnp.log(ut_ref[0] + _EPS)
    l1 = _EPS - jnp.log(ut_ref[1] + _EPS)
    x = (at_ref[0] - at_ref[1] + jnp.log(l1 / l0)) * (1.0 / _TEMP)
    o_ref[...] = jax.nn.sigmoid(x)


def kernel(gen_matrix, u):
    u3 = u.reshape(_SZ, _SZ, 2)
    at = jnp.moveaxis(gen_matrix, 2, 0)
    ut = jnp.moveaxis(u3, 2, 0)
    spec3 = pl.BlockSpec((2, _BR, _SZ), lambda i: (0, i, 0))
    return pl.pallas_call(
        _body,
        grid=(_SZ // _BR,),
        in_specs=[spec3, spec3],
        out_specs=pl.BlockSpec((_BR, _SZ), lambda i: (i, 0)),
        out_shape=jax.ShapeDtypeStruct((_SZ, _SZ), jnp.float32),
    )(at, ut)
